# baseline (speedup 1.0000x reference)
import jax
import jax.numpy as jnp
from jax.experimental import pallas as pl
from jax.experimental.pallas import tpu as pltpu

_N = 32768
_ROWS = 256
_COLS = 128
_HI_BITS = 0x3F7FFFFF


def _body(x_ref, k_ref, out_ref):
    x = x_ref[...]
    bits = jax.lax.bitcast_convert_type(x, jnp.int32)
    k = k_ref[0, 0]

    lo, hi = jnp.int32(0), jnp.int32(_HI_BITS)
    for _ in range(16):
        w = hi - lo + 1
        m1 = lo + jax.lax.shift_right_logical(w, 2)
        m2 = lo + jax.lax.shift_right_logical(w, 1)
        m3 = m1 + jax.lax.shift_right_logical(w, 1)
        c1 = jnp.sum((bits >= m1).astype(jnp.int32))
        c2 = jnp.sum((bits >= m2).astype(jnp.int32))
        c3 = jnp.sum((bits >= m3).astype(jnp.int32))
        sel = ((c1 >= k).astype(jnp.int32) + (c2 >= k).astype(jnp.int32)
               + (c3 >= k).astype(jnp.int32))
        lo = jnp.where(sel == 3, m3,
             jnp.where(sel == 2, m2, jnp.where(sel == 1, m1, lo)))
        hi = jnp.where(sel == 3, hi,
             jnp.where(sel == 2, m3 - 1,
             jnp.where(sel == 1, m2 - 1, m1 - 1)))
    t_bits = lo
    t = jax.lax.bitcast_convert_type(t_bits, jnp.float32)

    mask_gt = bits > t_bits
    y = jnp.where(mask_gt, x, 1.0 - x)
    s = jnp.sum(-jnp.log(y))
    c_gt = jnp.sum(mask_gt.astype(jnp.int32))
    g_t = jnp.log(1.0 - t) - jnp.log(t)
    total = s + (k - c_gt).astype(jnp.float32) * g_t
    out_ref[0, 0] = total / jnp.float32(_N)


def kernel(logits, pos_ratio):
    k = jnp.round(pos_ratio.reshape(()) * _N).astype(jnp.int32).reshape(1, 1)
    x = logits.reshape(_ROWS, _COLS)
    out = pl.pallas_call(
        _body,
        out_shape=jax.ShapeDtypeStruct((1, 1), jnp.float32),
        in_specs=[
            pl.BlockSpec(memory_space=pltpu.VMEM),
            pl.BlockSpec(memory_space=pltpu.SMEM),
        ],
        out_specs=pl.BlockSpec(memory_space=pltpu.SMEM),
    )(x, k)
    return out.reshape(())


# speedup vs baseline: 1.4343x; 1.4343x over previous
import jax
import jax.numpy as jnp
from jax.experimental import pallas as pl
from jax.experimental.pallas import tpu as pltpu

_N = 32768
_ROWS = 256
_COLS = 128
_HI_BITS = 0x3F7FFFFF


def _body(x_ref, k_ref, out_ref):
    x = x_ref[...]
    bits = jax.lax.bitcast_convert_type(x, jnp.int32)
    k = k_ref[0, 0]

    kf = k.astype(jnp.float32)

    def cond(state):
        lo, hi, a, b = state
        c_open = jnp.sum(((x > lo) & (x < hi)).astype(jnp.int32))
        return c_open > 0

    def body(state):
        lo, hi, a, b = state
        m = lo + (hi - lo) * ((a - kf) / (a - b))
        lo_b = jax.lax.bitcast_convert_type(lo, jnp.int32)
        hi_b = jax.lax.bitcast_convert_type(hi, jnp.int32)
        m_b = jax.lax.bitcast_convert_type(m, jnp.int32)
        m_b = jnp.minimum(jnp.maximum(m_b, lo_b + 1), hi_b - 1)
        m = jax.lax.bitcast_convert_type(m_b, jnp.float32)
        ge = x >= m
        c = jnp.sum(ge.astype(jnp.int32)).astype(jnp.float32)
        snap = jnp.min(jnp.where(ge, x, jnp.float32(2.0)))
        take = c >= kf
        return (jnp.where(take, snap, lo), jnp.where(take, hi, m),
                jnp.where(take, c, a), jnp.where(take, b, c))

    init = (jnp.float32(0.0), jnp.float32(1.0),
            jnp.float32(_N), jnp.float32(0.0))
    t, _, _, _ = jax.lax.while_loop(cond, body, init)
    t_bits = jax.lax.bitcast_convert_type(t, jnp.int32)
    t = jax.lax.bitcast_convert_type(t_bits, jnp.float32)

    mask_gt = bits > t_bits
    y = jnp.where(mask_gt, x, 1.0 - x)
    s = jnp.sum(-jnp.log(y))
    c_gt = jnp.sum(mask_gt.astype(jnp.int32))
    g_t = jnp.log(1.0 - t) - jnp.log(t)
    total = s + (k - c_gt).astype(jnp.float32) * g_t
    out_ref[0, 0] = total / jnp.float32(_N)


def kernel(logits, pos_ratio):
    k = jnp.round(pos_ratio.reshape(()) * _N).astype(jnp.int32).reshape(1, 1)
    x = logits.reshape(_ROWS, _COLS)
    out = pl.pallas_call(
        _body,
        out_shape=jax.ShapeDtypeStruct((1, 1), jnp.float32),
        in_specs=[
            pl.BlockSpec(memory_space=pltpu.VMEM),
            pl.BlockSpec(memory_space=pltpu.SMEM),
        ],
        out_specs=pl.BlockSpec(memory_space=pltpu.SMEM),
    )(x, k)
    return out.reshape(())
